# baseline (speedup 1.0000x reference)
import functools

import jax
import jax.numpy as jnp
from jax import lax
from jax.experimental import pallas as pl
from jax.experimental.pallas import tpu as pltpu
from jax.experimental.pallas import tpu_sc as plsc

NUM_EXPERTS = 8
TOP_K = 2
HIDDEN = 1024
INTER = 512
TOKENS = 2048
SLOTS = TOKENS * TOP_K
TM = 512
M_TILES = SLOTS // TM
G_STEPS = M_TILES + NUM_EXPERTS

NW = 32
SPW = SLOTS // NW
TPW = TOKENS // NW

_NEG = -1e30


def _route_plan_body(hidden_ref, gate_w_ref, bias_ref,
                     out1_ref, out2_ref, out3_ref):
    x = hidden_ref[...]
    logits_te = jax.lax.dot_general(
        x, gate_w_ref[...], (((1,), (1,)), ((), ())),
        preferred_element_type=jnp.float32)
    TB_ = 512
    lt_blocks = []
    for c in range(TOKENS // TB_):
        rr = jax.lax.broadcasted_iota(jnp.int32, (TOKENS, TB_), 0)
        uu = jax.lax.broadcasted_iota(jnp.int32, (TOKENS, TB_), 1) + c * TB_
        idb = (rr == uu).astype(jnp.float32)
        lt_blocks.append(jax.lax.dot_general(
            logits_te, idb, (((0,), (0,)), ((), ())),
            preferred_element_type=jnp.float32,
            precision=jax.lax.Precision.HIGHEST))
    logits = jnp.concatenate(lt_blocks, axis=1)
    s = jax.nn.sigmoid(logits)
    sc = s + bias_ref[...]
    e8 = jax.lax.broadcasted_iota(jnp.int32, (NUM_EXPERTS, TOKENS), 0)
    m0 = jnp.max(sc, axis=0, keepdims=True)
    i0 = jnp.min(jnp.where(sc == m0, e8, NUM_EXPERTS), axis=0, keepdims=True)
    oh0 = (e8 == i0)
    sc2 = jnp.where(oh0, _NEG, sc)
    m1 = jnp.max(sc2, axis=0, keepdims=True)
    i1 = jnp.min(jnp.where(sc2 == m1, e8, NUM_EXPERTS), axis=0, keepdims=True)
    oh1 = (e8 == i1)
    w0 = jnp.sum(jnp.where(oh0, s, 0.0), axis=0, keepdims=True)
    w1 = jnp.sum(jnp.where(oh1, s, 0.0), axis=0, keepdims=True)
    norm = w0 + w1 + 1e-6
    w0n = w0 / norm
    w1n = w1 / norm

    oh0f = oh0.astype(jnp.float32)
    oh1f = oh1.astype(jnp.float32)

    CB = 512
    r0_blocks, r1_blocks = [], []
    for c in range(TOKENS // CB):
        tp = jax.lax.broadcasted_iota(jnp.int32, (TOKENS, CB), 0)
        tc = jax.lax.broadcasted_iota(jnp.int32, (TOKENS, CB), 1) + c * CB
        u = (tp < tc).astype(jnp.float32)
        r0_blocks.append(jax.lax.dot_general(
            oh0f, u, (((1,), (0,)), ((), ())),
            preferred_element_type=jnp.float32))
        r1_blocks.append(jax.lax.dot_general(
            oh1f, u, (((1,), (0,)), ((), ())),
            preferred_element_type=jnp.float32))
    r0 = jnp.concatenate(r0_blocks, axis=1)
    r1 = jnp.concatenate(r1_blocks, axis=1)

    c0 = jnp.sum(oh0f, axis=1, keepdims=True)
    c1 = jnp.sum(oh1f, axis=1, keepdims=True)
    cnt = c0 + c1
    er = jax.lax.broadcasted_iota(jnp.int32, (NUM_EXPERTS, NUM_EXPERTS), 0)
    ec = jax.lax.broadcasted_iota(jnp.int32, (NUM_EXPERTS, NUM_EXPERTS), 1)
    lt = (ec < er).astype(jnp.float32)
    le = (ec <= er).astype(jnp.float32)
    offs = jax.lax.dot_general(lt, cnt, (((1,), (0,)), ((), ())),
                               preferred_element_type=jnp.float32,
                               precision=jax.lax.Precision.HIGHEST)

    pos0 = jnp.sum(oh0f * (offs + r0), axis=0, keepdims=True)
    pos1 = jnp.sum(oh1f * (offs + c0 + r1), axis=0, keepdims=True)
    pad = jnp.zeros((4, TOKENS), jnp.float32)
    out1_ref[...] = jnp.concatenate([w0n, w1n, pos0, pos1, pad], axis=0)

    tmf = float(TM)
    start = offs
    end = offs + cnt
    t_lo = jnp.minimum(jnp.floor(start / tmf), float(M_TILES - 1))
    t_hi = jnp.maximum(t_lo, jnp.ceil(end / tmf) - 1.0)
    c_e = jnp.where(cnt > 0, t_hi - t_lo + 1.0, 1.0)
    cumc = jax.lax.dot_general(le, c_e, (((1,), (0,)), ((), ())),
                               preferred_element_type=jnp.float32,
                               precision=jax.lax.Precision.HIGHEST)
    cumx = cumc - c_e
    gf = jax.lax.broadcasted_iota(jnp.int32, (1, 128), 1).astype(jnp.float32)
    gid = jnp.minimum(
        jnp.sum((cumc <= gf).astype(jnp.float32), axis=0, keepdims=True),
        float(NUM_EXPERTS - 1))
    e81 = jax.lax.broadcasted_iota(jnp.int32, (NUM_EXPERTS, 128), 0)
    ohg = (e81 == gid.astype(jnp.int32)).astype(jnp.float32)
    t_lo_g = jnp.sum(ohg * t_lo, axis=0, keepdims=True)
    exc_g = jnp.sum(ohg * cumx, axis=0, keepdims=True)
    ptot = jnp.sum(c_e)
    valid = (gf < ptot).astype(jnp.float32)
    mt = jnp.where(valid > 0, t_lo_g + (gf - exc_g), float(M_TILES - 1))
    gidv = jnp.where(valid > 0, gid, float(NUM_EXPERTS - 1))
    offs_row = (jnp.sum((e81 == gf.astype(jnp.int32)).astype(jnp.float32)
                        * start, axis=0, keepdims=True)
                + (gf == float(NUM_EXPERTS)) * float(SLOTS))
    pad2 = jnp.zeros((4, 128), jnp.float32)
    out2_ref[...] = jnp.concatenate(
        [offs_row, gidv, mt, valid, pad2], axis=0).astype(jnp.int32)

    ws = jnp.concatenate([w0n, w1n, jnp.zeros((6, TOKENS), jnp.float32)],
                         axis=0)
    RB = 512
    col_blocks = []
    for b in range(TOKENS // RB):
        rb = jax.lax.broadcasted_iota(jnp.int32, (RB, TOKENS), 0) + b * RB
        cb = jax.lax.broadcasted_iota(jnp.int32, (RB, TOKENS), 1)
        idb = (rb == cb).astype(jnp.float32)
        col_blocks.append(jax.lax.dot_general(
            idb, ws, (((1,), (1,)), ((), ())),
            preferred_element_type=jnp.float32,
            precision=jax.lax.Precision.HIGHEST))
    out3_ref[...] = jnp.concatenate(col_blocks, axis=0)


def _route_plan(hidden_states, gate_w, expert_bias):
    return pl.pallas_call(
        _route_plan_body,
        out_shape=[
            jax.ShapeDtypeStruct((8, TOKENS), jnp.float32),
            jax.ShapeDtypeStruct((8, 128), jnp.int32),
            jax.ShapeDtypeStruct((TOKENS, NUM_EXPERTS), jnp.float32),
        ],
    )(hidden_states, gate_w, expert_bias.reshape(NUM_EXPERTS, 1))


def _sc_mesh():
    return plsc.VectorSubcoreMesh(core_axis_name="c", subcore_axis_name="s")


def _scatter_sorted(pos, hidden_states):
    @functools.partial(
        pl.kernel,
        mesh=_sc_mesh(),
        out_type=jax.ShapeDtypeStruct((SLOTS, HIDDEN), jnp.float32),
        scratch_types=[
            pltpu.VMEM((64,), jnp.int32),
            pltpu.VMEM((64,), jnp.int32),
            pltpu.VMEM((64, HIDDEN), jnp.float32),
            pltpu.SemaphoreType.DMA,
        ],
    )
    def k2(pos_hbm, hid_hbm, xs_hbm, p_a, p_b, rows_v, sem):
        wid = lax.axis_index("s") * 2 + lax.axis_index("c")
        jb = pl.multiple_of(wid * SPW, SPW)
        t0 = pl.multiple_of(jb & (TOKENS - 1), 64)
        pltpu.sync_copy(pos_hbm.at[pl.ds(jb, 64)], p_a)
        pltpu.sync_copy(pos_hbm.at[pl.ds(jb + 64, 64)], p_b)
        pltpu.sync_copy(hid_hbm.at[pl.ds(t0, 64)], rows_v)
        pltpu.async_copy(rows_v, xs_hbm.at[p_a], sem).wait()
        pltpu.sync_copy(hid_hbm.at[pl.ds(t0 + 64, 64)], rows_v)
        pltpu.async_copy(rows_v, xs_hbm.at[p_b], sem).wait()

    return k2(pos, hidden_states)


def _gmm_body(mt_ref, gid_ref, off_ref, val_ref,
              xs_ref, gup_ref, down_ref, y_ref):
    g = pl.program_id(0)
    e = gid_ref[g]
    mt = mt_ref[g]
    x = xs_ref[...].astype(jnp.bfloat16)
    gu = jax.lax.dot_general(
        x, gup_ref[0].astype(jnp.bfloat16), (((1,), (1,)), ((), ())),
        preferred_element_type=jnp.float32)
    gate = gu[:, :INTER]
    up = gu[:, INTER:]
    act = (gate * jax.nn.sigmoid(gate)) * up
    r = jax.lax.broadcasted_iota(jnp.int32, (TM, 1), 0) + mt * TM
    keep = (r >= off_ref[e]) & (r < off_ref[e + 1]) & (val_ref[g] > 0)
    act = (act * keep.astype(jnp.float32)).astype(jnp.bfloat16)
    eo = jax.lax.dot_general(
        act, down_ref[0].astype(jnp.bfloat16), (((1,), (1,)), ((), ())),
        preferred_element_type=jnp.float32)
    first = (g == 0) | (mt != mt_ref[jnp.maximum(g - 1, 0)])
    y_ref[...] = jnp.where(first, 0.0, y_ref[...]) + eo


def _grouped_mlp(mt, gid, off, valid, xs, gate_up_proj, down_proj):
    grid_spec = pltpu.PrefetchScalarGridSpec(
        num_scalar_prefetch=4,
        grid=(G_STEPS,),
        in_specs=[
            pl.BlockSpec((TM, HIDDEN), lambda g, mt, gid, off, val: (mt[g], 0)),
            pl.BlockSpec((1, 2 * INTER, HIDDEN),
                         lambda g, mt, gid, off, val: (gid[g], 0, 0)),
            pl.BlockSpec((1, HIDDEN, INTER),
                         lambda g, mt, gid, off, val: (gid[g], 0, 0)),
        ],
        out_specs=pl.BlockSpec((TM, HIDDEN),
                               lambda g, mt, gid, off, val: (mt[g], 0)),
    )
    return pl.pallas_call(
        _gmm_body,
        grid_spec=grid_spec,
        out_shape=jax.ShapeDtypeStruct((SLOTS, HIDDEN), jnp.float32),
    )(mt, gid, off, valid, xs, gate_up_proj, down_proj)


def _gather_pair(pos, y):
    @functools.partial(
        pl.kernel,
        mesh=_sc_mesh(),
        out_type=[
            jax.ShapeDtypeStruct((TOKENS, HIDDEN), jnp.float32),
            jax.ShapeDtypeStruct((TOKENS, HIDDEN), jnp.float32),
        ],
        scratch_types=[
            pltpu.VMEM((64,), jnp.int32),
            pltpu.VMEM((64,), jnp.int32),
            pltpu.VMEM((64, HIDDEN), jnp.float32),
            pltpu.SemaphoreType.DMA,
        ],
    )
    def k4a(pos_hbm, y_hbm, a_hbm, b_hbm, p0_v, p1_v, rows_v, sem):
        wid = lax.axis_index("s") * 2 + lax.axis_index("c")
        t0 = pl.multiple_of(wid * TPW, TPW)
        pltpu.sync_copy(pos_hbm.at[pl.ds(t0, 64)], p0_v)
        pltpu.sync_copy(pos_hbm.at[pl.ds(TOKENS + t0, 64)], p1_v)
        pltpu.async_copy(y_hbm.at[p0_v], rows_v, sem).wait()
        pltpu.sync_copy(rows_v, a_hbm.at[pl.ds(t0, 64)])
        pltpu.async_copy(y_hbm.at[p1_v], rows_v, sem).wait()
        pltpu.sync_copy(rows_v, b_hbm.at[pl.ds(t0, 64)])

    return k4a(pos, y)


def _combine_body(a_ref, b_ref, w_ref, out_ref):
    w = w_ref[...]
    out_ref[...] = a_ref[...] * w[:, 0:1] + b_ref[...] * w[:, 1:2]


def _combine(a, b, wcols):
    TB = 256
    return pl.pallas_call(
        _combine_body,
        grid=(TOKENS // TB,),
        in_specs=[
            pl.BlockSpec((TB, HIDDEN), lambda i: (i, 0)),
            pl.BlockSpec((TB, HIDDEN), lambda i: (i, 0)),
            pl.BlockSpec((TB, NUM_EXPERTS), lambda i: (i, 0)),
        ],
        out_specs=pl.BlockSpec((TB, HIDDEN), lambda i: (i, 0)),
        out_shape=jax.ShapeDtypeStruct((TOKENS, HIDDEN), jnp.float32),
    )(a, b, wcols)


@jax.jit
def kernel(hidden_states, gate_w, expert_bias, gate_up_proj, down_proj):
    meta1, meta2, wcols = _route_plan(hidden_states, gate_w, expert_bias)
    pos = meta1[2:4].reshape(SLOTS).astype(jnp.int32)
    off = meta2[0, :16]
    gid = meta2[1, :G_STEPS]
    mt = meta2[2, :G_STEPS]
    valid = meta2[3, :G_STEPS]
    xs = _scatter_sorted(pos, hidden_states)
    y = _grouped_mlp(mt, gid, off, valid, xs, gate_up_proj, down_proj)
    a, b = _gather_pair(pos, y)
    return _combine(a, b, wcols)


# speedup vs baseline: 1.7207x; 1.7207x over previous
import jax
import jax.numpy as jnp
from jax.experimental import pallas as pl
from jax.experimental.pallas import tpu as pltpu

NUM_EXPERTS = 8
TOP_K = 2
HIDDEN = 1024
INTER = 512
TOKENS = 2048

_NEG = -1e30


def _route(x, gw, bias):
    logits = jax.lax.dot_general(
        x, gw, (((1,), (1,)), ((), ())), preferred_element_type=jnp.float32)
    s = jax.nn.sigmoid(logits)
    sc = s + bias
    e_iota = jax.lax.broadcasted_iota(jnp.int32, sc.shape, 1)
    m0 = jnp.max(sc, axis=1, keepdims=True)
    i0 = jnp.min(jnp.where(sc == m0, e_iota, NUM_EXPERTS), axis=1, keepdims=True)
    oh0 = (e_iota == i0)
    sc2 = jnp.where(oh0, _NEG, sc)
    m1 = jnp.max(sc2, axis=1, keepdims=True)
    i1 = jnp.min(jnp.where(sc2 == m1, e_iota, NUM_EXPERTS), axis=1, keepdims=True)
    oh1 = (e_iota == i1)
    w0 = jnp.sum(jnp.where(oh0, s, 0.0), axis=1, keepdims=True)
    w1 = jnp.sum(jnp.where(oh1, s, 0.0), axis=1, keepdims=True)
    norm = w0 + w1 + 1e-6
    return (jnp.where(oh0, s, 0.0) + jnp.where(oh1, s, 0.0)) / norm


def _moe_body(hidden_ref, gate_w_ref, bias_ref, gup_ref, down_ref,
              out_ref, w_ref, act_ref, xb_ref):
    e = pl.program_id(0)

    @pl.when(e == 0)
    def _do_route():
        w_ref[...] = _route(hidden_ref[...], gate_w_ref[...], bias_ref[...])
        xb_ref[...] = hidden_ref[...].astype(jnp.bfloat16)

    act_prev = act_ref[...]
    eo = jax.lax.dot_general(
        act_prev, down_ref[0].astype(jnp.bfloat16), (((1,), (1,)), ((), ())),
        preferred_element_type=jnp.float32)
    out_ref[...] = jnp.where(e >= 2, out_ref[...], 0.0) + eo

    wmat = w_ref[...]
    e_iota = jax.lax.broadcasted_iota(jnp.int32, wmat.shape, 1)
    wcol = jnp.sum(jnp.where(e_iota == e, wmat, 0.0), axis=1, keepdims=True)
    x = xb_ref[...]
    gu = jax.lax.dot_general(
        x, gup_ref[0].astype(jnp.bfloat16), (((1,), (1,)), ((), ())),
        preferred_element_type=jnp.float32)
    gate = gu[:, :INTER]
    up = gu[:, INTER:]
    act_ref[...] = ((gate * jax.nn.sigmoid(gate)) * up * wcol).astype(
        jnp.bfloat16)


@jax.jit
def kernel(hidden_states, gate_w, expert_bias, gate_up_proj, down_proj):
    out = pl.pallas_call(
        _moe_body,
        grid=(NUM_EXPERTS + 1,),
        in_specs=[
            pl.BlockSpec((TOKENS, HIDDEN), lambda e: (0, 0)),
            pl.BlockSpec((NUM_EXPERTS, HIDDEN), lambda e: (0, 0)),
            pl.BlockSpec((1, NUM_EXPERTS), lambda e: (0, 0)),
            pl.BlockSpec((1, 2 * INTER, HIDDEN),
                         lambda e: (jnp.minimum(e, NUM_EXPERTS - 1), 0, 0)),
            pl.BlockSpec((1, HIDDEN, INTER),
                         lambda e: (jnp.maximum(e - 1, 0), 0, 0)),
        ],
        out_specs=pl.BlockSpec((TOKENS, HIDDEN), lambda e: (0, 0)),
        out_shape=jax.ShapeDtypeStruct((TOKENS, HIDDEN), jnp.float32),
        scratch_shapes=[
            pltpu.VMEM((TOKENS, NUM_EXPERTS), jnp.float32),
            pltpu.VMEM((TOKENS, INTER), jnp.bfloat16),
            pltpu.VMEM((TOKENS, HIDDEN), jnp.bfloat16),
        ],
    )(hidden_states, gate_w, expert_bias.reshape(1, NUM_EXPERTS),
      gate_up_proj, down_proj)
    return out
